# baseline (speedup 1.0000x reference)
import jax
import jax.numpy as jnp
from jax.experimental import pallas as pl
from jax.experimental.pallas import tpu as pltpu

_K = 512
_D = 32
_BETA = 0.5
_KT = 64


def _combine8(q):
    return ((q[0] + q[4]) + (q[2] + q[6])) + ((q[1] + q[5]) + (q[3] + q[7]))


def _dist_tree(ps):
    cs = [_combine8(ps[8 * c:8 * c + 8]) for c in range(len(ps) // 8)]
    acc = cs[0]
    for c in cs[1:]:
        acc = acc + c
    return acc


def _vq_body(xT_ref, W_ref, WT_ref, xqT_ref, idx_ref, dsum_ref):
    nt = xT_ref.shape[2]
    xT = xT_ref[0]
    run_min = jnp.full((1, nt), jnp.inf, dtype=jnp.float32)
    run_idx = jnp.zeros((1, nt), dtype=jnp.int32)
    for t in range(_K // _KT):
        Wt = W_ref[t * _KT:(t + 1) * _KT, :]
        ps = []
        for d in range(_D):
            diff = Wt[:, d:d + 1] - xT[d:d + 1, :]
            ps.append(diff * diff)
        dist = _dist_tree(ps)
        tmin = jnp.min(dist, axis=0, keepdims=True)
        kio = jax.lax.broadcasted_iota(jnp.int32, (_KT, nt), 0)
        tidx = jnp.min(jnp.where(dist == tmin, kio, _K),
                       axis=0, keepdims=True) + t * _KT
        better = tmin < run_min
        run_min = jnp.where(better, tmin, run_min)
        run_idx = jnp.where(better, tidx, run_idx)
    idx_ref[0, 0, :] = run_idx[0]
    dsum_ref[0] = jnp.sum(run_min, axis=(0, 1), keepdims=True)
    onehot = (jax.lax.broadcasted_iota(jnp.int32, (_K, nt), 0)
              == run_idx).astype(jnp.float32)
    xqT = jax.lax.dot_general(
        WT_ref[...], onehot, (((1,), (0,)), ((), ())),
        preferred_element_type=jnp.float32,
        precision=jax.lax.Precision.HIGHEST)
    xqT_ref[0] = xT + (xqT - xT)


def kernel(x, W):
    b, d, h, w = x.shape
    nt = h * w
    xT = x.reshape(b, d, nt)
    WT = W.T
    xqT, idx, dsum = pl.pallas_call(
        _vq_body,
        grid=(b,),
        in_specs=[
            pl.BlockSpec((1, d, nt), lambda i: (i, 0, 0)),
            pl.BlockSpec((_K, _D), lambda i: (0, 0)),
            pl.BlockSpec((_D, _K), lambda i: (0, 0)),
        ],
        out_specs=[
            pl.BlockSpec((1, d, nt), lambda i: (i, 0, 0)),
            pl.BlockSpec((1, 1, nt), lambda i: (i, 0, 0)),
            pl.BlockSpec((1, 1, 1), lambda i: (i, 0, 0)),
        ],
        out_shape=[
            jax.ShapeDtypeStruct((b, d, nt), jnp.float32),
            jax.ShapeDtypeStruct((b, 1, nt), jnp.int32),
            jax.ShapeDtypeStruct((b, 1, 1), jnp.float32),
        ],
        compiler_params=pltpu.CompilerParams(
            dimension_semantics=("parallel",)),
    )(xT, W, WT)
    x_q_st = xqT.reshape(b, d, h, w)
    latent_indices = idx.reshape(b * h * w)
    loss = (1.0 + _BETA) * jnp.sum(dsum) / (b * d * h * w)
    return (x_q_st, loss, latent_indices)


# speedup vs baseline: 1.1690x; 1.1690x over previous
import jax
import jax.numpy as jnp
from jax.experimental import pallas as pl
from jax.experimental.pallas import tpu as pltpu

_K = 512
_D = 32
_BETA = 0.5
_KT = 64


def _combine8(q):
    return ((q[0] + q[4]) + (q[2] + q[6])) + ((q[1] + q[5]) + (q[3] + q[7]))


def _dist_tree(ps):
    cs = [_combine8(ps[8 * c:8 * c + 8]) for c in range(len(ps) // 8)]
    acc = cs[0]
    for c in cs[1:]:
        acc = acc + c
    return acc


def _vq_body(x_ref, W_ref, xq_ref, idx_ref, loss_ref):
    nb = pl.num_programs(0)
    i = pl.program_id(0)
    dd, hh, ww = x_ref.shape[1:]
    nt = hh * ww
    xT = x_ref[0].reshape(dd, nt)
    run_min = jnp.full((1, nt), jnp.inf, dtype=jnp.float32)
    run_idx = jnp.zeros((1, nt), dtype=jnp.int32)
    for t in range(_K // _KT):
        Wt = W_ref[t * _KT:(t + 1) * _KT, :]
        ps = []
        for d in range(_D):
            diff = Wt[:, d:d + 1] - xT[d:d + 1, :]
            ps.append(diff * diff)
        dist = _dist_tree(ps)
        tmin = jnp.min(dist, axis=0, keepdims=True)
        kio = jax.lax.broadcasted_iota(jnp.int32, (_KT, nt), 0)
        tidx = jnp.min(jnp.where(dist == tmin, kio, _K),
                       axis=0, keepdims=True) + t * _KT
        better = tmin < run_min
        run_min = jnp.where(better, tmin, run_min)
        run_idx = jnp.where(better, tidx, run_idx)
    idx_ref[0, :] = run_idx[0]
    onehot = (jax.lax.broadcasted_iota(jnp.int32, (_K, nt), 0)
              == run_idx).astype(jnp.float32)
    xqT = jax.lax.dot_general(
        W_ref[...], onehot, (((0,), (0,)), ((), ())),
        preferred_element_type=jnp.float32,
        precision=jax.lax.Precision.HIGHEST)
    xq_ref[0] = (xT + (xqT - xT)).reshape(dd, hh, ww)
    part = jnp.sum(run_min, axis=(0, 1), keepdims=True)

    @pl.when(i == 0)
    def _init():
        loss_ref[...] = jnp.zeros_like(loss_ref)

    loss_ref[...] += part

    @pl.when(i == nb - 1)
    def _fini():
        loss_ref[...] *= (1.0 + _BETA) / (nb * dd * nt)


def kernel(x, W):
    b, d, h, w = x.shape
    nt = h * w
    xq, idx, loss = pl.pallas_call(
        _vq_body,
        grid=(b,),
        in_specs=[
            pl.BlockSpec((1, d, h, w), lambda i: (i, 0, 0, 0)),
            pl.BlockSpec((_K, _D), lambda i: (0, 0)),
        ],
        out_specs=[
            pl.BlockSpec((1, d, h, w), lambda i: (i, 0, 0, 0)),
            pl.BlockSpec((1, nt), lambda i: (0, i)),
            pl.BlockSpec((1, 1), lambda i: (0, 0)),
        ],
        out_shape=[
            jax.ShapeDtypeStruct((b, d, h, w), jnp.float32),
            jax.ShapeDtypeStruct((1, b * nt), jnp.int32),
            jax.ShapeDtypeStruct((1, 1), jnp.float32),
        ],
    )(x, W)
    return (xq, loss[0, 0], idx.reshape(b * nt))
